# initial kernel scaffold (unmeasured)
import jax
import jax.numpy as jnp
from jax import lax
from jax.experimental import pallas as pl
from jax.experimental.pallas import tpu as pltpu

N_DEV = 4
M_PER = 2048
D = 2048
EPS = 1e-6


def kernel(partial, gamma):
    gamma2d = gamma.reshape(1, D)

    def body(p_ref, g_ref, o_ref, recv_ref, acc_ref, loc_ref,
             send_sem, recv_sems, dma_sem):
        my = lax.axis_index("i")
        left = lax.rem(my + N_DEV - 1, N_DEV)
        right = lax.rem(my + 1, N_DEV)

        barrier = pltpu.get_barrier_semaphore()
        for nbr in (left, right):
            pl.semaphore_signal(
                barrier, inc=1,
                device_id=(nbr,), device_id_type=pl.DeviceIdType.MESH,
            )
        pl.semaphore_wait(barrier, 2)

        def load_chunk(c, dst):
            cp = pltpu.make_async_copy(
                p_ref.at[0, pl.ds(c * M_PER, M_PER), :], dst, dma_sem)
            cp.start()
            cp.wait()

        load_chunk(left, acc_ref)

        for h in range(N_DEV - 1):
            rdma = pltpu.make_async_remote_copy(
                src_ref=acc_ref,
                dst_ref=recv_ref.at[h],
                send_sem=send_sem,
                recv_sem=recv_sems.at[h],
                device_id=(right,),
                device_id_type=pl.DeviceIdType.MESH,
            )
            rdma.start()
            rdma.wait()
            c = lax.rem(my + 2 * N_DEV - 2 - h, N_DEV)
            load_chunk(c, loc_ref)
            acc_ref[...] = recv_ref[h] + loc_ref[...]

        y = acc_ref[...]
        rms = jnp.sqrt(jnp.mean(y * y, axis=-1, keepdims=True) + EPS)
        o_ref[...] = y / rms * g_ref[...]

    return pl.pallas_call(
        body,
        out_shape=jax.ShapeDtypeStruct((M_PER, D), jnp.float32),
        in_specs=[
            pl.BlockSpec(memory_space=pltpu.ANY),
            pl.BlockSpec(memory_space=pltpu.VMEM),
        ],
        out_specs=pl.BlockSpec(memory_space=pltpu.VMEM),
        scratch_shapes=[
            pltpu.VMEM((N_DEV - 1, M_PER, D), jnp.float32),
            pltpu.VMEM((M_PER, D), jnp.float32),
            pltpu.VMEM((M_PER, D), jnp.float32),
            pltpu.SemaphoreType.DMA,
            pltpu.SemaphoreType.DMA((N_DEV - 1,)),
            pltpu.SemaphoreType.DMA,
        ],
        compiler_params=pltpu.CompilerParams(collective_id=0),
    )(partial, gamma2d)


# baseline (device time: 611600 ns/iter reference)
import jax
import jax.numpy as jnp
from jax import lax
from jax.experimental import pallas as pl
from jax.experimental.pallas import tpu as pltpu

N_DEV = 4
M_PER = 2048
D = 2048
EPS = 1e-6


def kernel(partial, gamma):
    gamma2d = gamma.reshape(1, D)

    def body(p_ref, g_ref, o_ref, recv_ref, acc_ref, tmp_ref,
             send_sem, recv_sems, dma_sem1, dma_sem2):
        my = lax.axis_index("i")
        left = lax.rem(my + N_DEV - 1, N_DEV)
        right = lax.rem(my + 1, N_DEV)

        barrier = pltpu.get_barrier_semaphore()
        for nbr in (left, right):
            pl.semaphore_signal(
                barrier, inc=1,
                device_id=(nbr,), device_id_type=pl.DeviceIdType.MESH,
            )
        pl.semaphore_wait(barrier, 2)

        def chunk(c):
            return p_ref.at[0, pl.ds(c * M_PER, M_PER), :]

        cp = pltpu.make_async_copy(chunk(left), acc_ref, dma_sem1)
        cp.start()
        cp.wait()

        for h in range(N_DEV - 1):
            rdma = pltpu.make_async_remote_copy(
                src_ref=acc_ref,
                dst_ref=recv_ref.at[h],
                send_sem=send_sem,
                recv_sem=recv_sems.at[h],
                device_id=(right,),
                device_id_type=pl.DeviceIdType.MESH,
            )
            rdma.start()
            rdma.wait()
            c = lax.rem(my + 2 * N_DEV - 2 - h, N_DEV)
            cp1 = pltpu.make_async_copy(recv_ref.at[h], acc_ref, dma_sem1)
            cp2 = pltpu.make_async_copy(chunk(c), tmp_ref, dma_sem2)
            cp1.start()
            cp2.start()
            cp1.wait()
            cp2.wait()
            acc_ref[...] = acc_ref[...] + tmp_ref[...]

        y = acc_ref[...]
        rms = jnp.sqrt(jnp.mean(y * y, axis=-1, keepdims=True) + EPS)
        acc_ref[...] = y / rms * g_ref[...]
        cpo = pltpu.make_async_copy(acc_ref, o_ref, dma_sem1)
        cpo.start()
        cpo.wait()

    out, _ = pl.pallas_call(
        body,
        out_shape=(
            jax.ShapeDtypeStruct((M_PER, D), jnp.float32),
            jax.ShapeDtypeStruct((N_DEV - 1, M_PER, D), jnp.float32),
        ),
        in_specs=[
            pl.BlockSpec(memory_space=pl.ANY),
            pl.BlockSpec(memory_space=pltpu.VMEM),
        ],
        out_specs=(
            pl.BlockSpec(memory_space=pl.ANY),
            pl.BlockSpec(memory_space=pl.ANY),
        ),
        scratch_shapes=[
            pltpu.VMEM((M_PER, D), jnp.float32),
            pltpu.VMEM((M_PER, D), jnp.float32),
            pltpu.SemaphoreType.DMA,
            pltpu.SemaphoreType.DMA((N_DEV - 1,)),
            pltpu.SemaphoreType.DMA,
            pltpu.SemaphoreType.DMA,
        ],
        compiler_params=pltpu.CompilerParams(
            collective_id=0, vmem_limit_bytes=63 * 1024 * 1024),
    )(partial, gamma2d)
    return out


# device time: 327612 ns/iter; 1.8668x vs baseline; 1.8668x over previous
import jax
import jax.numpy as jnp
from jax import lax
from jax.experimental import pallas as pl
from jax.experimental.pallas import tpu as pltpu

N_DEV = 4
M_PER = 2048
D = 2048
DH = D // 2
EPS = 1e-6


def kernel(partial, gamma):
    gamma2d = gamma.reshape(1, D)

    def body(p_ref, g_ref, o_ref, stage_ref,
             acc_cw, acc_ccw, tmp_cw, tmp_ccw,
             send_sems, recv_sems, dsems):
        my = lax.axis_index("i")
        left = lax.rem(my + N_DEV - 1, N_DEV)
        right = lax.rem(my + 1, N_DEV)

        barrier = pltpu.get_barrier_semaphore()
        for nbr in (left, right):
            pl.semaphore_signal(
                barrier, inc=1,
                device_id=(nbr,), device_id_type=pl.DeviceIdType.MESH,
            )
        pl.semaphore_wait(barrier, 2)

        def chunk_lo(c):
            return p_ref.at[0, pl.ds(c * M_PER, M_PER), pl.ds(0, DH)]

        def chunk_hi(c):
            return p_ref.at[0, pl.ds(c * M_PER, M_PER), pl.ds(DH, DH)]

        s0 = pltpu.make_async_copy(chunk_lo(left), acc_cw, dsems.at[0])
        s1 = pltpu.make_async_copy(chunk_hi(right), acc_ccw, dsems.at[1])
        s0.start()
        s1.start()
        s0.wait()
        s1.wait()

        for h in range(N_DEV - 1):
            rdma_cw = pltpu.make_async_remote_copy(
                src_ref=acc_cw,
                dst_ref=stage_ref.at[0, h],
                send_sem=send_sems.at[0],
                recv_sem=recv_sems.at[0, h],
                device_id=(right,),
                device_id_type=pl.DeviceIdType.MESH,
            )
            rdma_ccw = pltpu.make_async_remote_copy(
                src_ref=acc_ccw,
                dst_ref=stage_ref.at[1, h],
                send_sem=send_sems.at[1],
                recv_sem=recv_sems.at[1, h],
                device_id=(left,),
                device_id_type=pl.DeviceIdType.MESH,
            )
            rdma_cw.start()
            rdma_ccw.start()

            c_cw = lax.rem(my + 2 * N_DEV - 2 - h, N_DEV)
            c_ccw = lax.rem(my + 2 + h, N_DEV)
            p0 = pltpu.make_async_copy(chunk_lo(c_cw), tmp_cw, dsems.at[0])
            p1 = pltpu.make_async_copy(chunk_hi(c_ccw), tmp_ccw, dsems.at[1])
            p0.start()
            p1.start()

            rdma_cw.wait()
            rdma_ccw.wait()

            r0 = pltpu.make_async_copy(stage_ref.at[0, h], acc_cw, dsems.at[2])
            r1 = pltpu.make_async_copy(stage_ref.at[1, h], acc_ccw, dsems.at[3])
            r0.start()
            r1.start()
            p0.wait()
            p1.wait()
            r0.wait()
            r1.wait()

            acc_cw[...] = acc_cw[...] + tmp_cw[...]
            acc_ccw[...] = acc_ccw[...] + tmp_ccw[...]

        ssq = (jnp.sum(acc_cw[...] * acc_cw[...], axis=-1, keepdims=True)
               + jnp.sum(acc_ccw[...] * acc_ccw[...], axis=-1, keepdims=True))
        inv = lax.rsqrt(ssq / D + EPS)
        acc_cw[...] = acc_cw[...] * inv * g_ref[:, 0:DH]
        acc_ccw[...] = acc_ccw[...] * inv * g_ref[:, DH:D]

        o0 = pltpu.make_async_copy(acc_cw, o_ref.at[:, pl.ds(0, DH)],
                                   dsems.at[0])
        o1 = pltpu.make_async_copy(acc_ccw, o_ref.at[:, pl.ds(DH, DH)],
                                   dsems.at[1])
        o0.start()
        o1.start()
        o0.wait()
        o1.wait()

    out, _ = pl.pallas_call(
        body,
        out_shape=(
            jax.ShapeDtypeStruct((M_PER, D), jnp.float32),
            jax.ShapeDtypeStruct((2, N_DEV - 1, M_PER, DH), jnp.float32),
        ),
        in_specs=[
            pl.BlockSpec(memory_space=pl.ANY),
            pl.BlockSpec(memory_space=pltpu.VMEM),
        ],
        out_specs=(
            pl.BlockSpec(memory_space=pl.ANY),
            pl.BlockSpec(memory_space=pl.ANY),
        ),
        scratch_shapes=[
            pltpu.VMEM((M_PER, DH), jnp.float32),
            pltpu.VMEM((M_PER, DH), jnp.float32),
            pltpu.VMEM((M_PER, DH), jnp.float32),
            pltpu.VMEM((M_PER, DH), jnp.float32),
            pltpu.SemaphoreType.DMA((2,)),
            pltpu.SemaphoreType.DMA((2, N_DEV - 1)),
            pltpu.SemaphoreType.DMA((4,)),
        ],
        compiler_params=pltpu.CompilerParams(
            collective_id=0, vmem_limit_bytes=63 * 1024 * 1024),
    )(partial, gamma2d)
    return out


# device time: 309692 ns/iter; 1.9749x vs baseline; 1.0579x over previous
import jax
import jax.numpy as jnp
from jax import lax
from jax.experimental import pallas as pl
from jax.experimental.pallas import tpu as pltpu

N_DEV = 4
M_PER = 2048
D = 2048
DH = D // 2
EPS = 1e-6


def kernel(partial, gamma):
    gamma2d = gamma.reshape(1, D)

    def body(p_ref, g_ref, o_ref,
             cw_slots, ccw_slots, tmp_cw, tmp_ccw,
             send_sems, recv_sems, dsems, credit_cw, credit_ccw):
        my = lax.axis_index("i")
        left = lax.rem(my + N_DEV - 1, N_DEV)
        right = lax.rem(my + 1, N_DEV)

        barrier = pltpu.get_barrier_semaphore()
        for nbr in (left, right):
            pl.semaphore_signal(
                barrier, inc=1,
                device_id=(nbr,), device_id_type=pl.DeviceIdType.MESH,
            )
        pl.semaphore_wait(barrier, 2)

        def chunk_lo(c):
            return p_ref.at[0, pl.ds(c * M_PER, M_PER), pl.ds(0, DH)]

        def chunk_hi(c):
            return p_ref.at[0, pl.ds(c * M_PER, M_PER), pl.ds(DH, DH)]

        s0 = pltpu.make_async_copy(chunk_lo(left), cw_slots.at[0], dsems.at[0])
        s1 = pltpu.make_async_copy(chunk_hi(right), ccw_slots.at[0],
                                   dsems.at[1])
        s0.start()
        s1.start()
        s0.wait()
        s1.wait()

        for h in range(N_DEV - 1):
            s = h % 2
            r = (h + 1) % 2
            if h >= 1:
                pl.semaphore_wait(credit_cw, 1)
                pl.semaphore_wait(credit_ccw, 1)

            rdma_cw = pltpu.make_async_remote_copy(
                src_ref=cw_slots.at[s],
                dst_ref=cw_slots.at[r],
                send_sem=send_sems.at[0],
                recv_sem=recv_sems.at[0, r],
                device_id=(right,),
                device_id_type=pl.DeviceIdType.MESH,
            )
            rdma_ccw = pltpu.make_async_remote_copy(
                src_ref=ccw_slots.at[s],
                dst_ref=ccw_slots.at[r],
                send_sem=send_sems.at[1],
                recv_sem=recv_sems.at[1, r],
                device_id=(left,),
                device_id_type=pl.DeviceIdType.MESH,
            )
            rdma_cw.start()
            rdma_ccw.start()

            c_cw = lax.rem(my + 2 * N_DEV - 2 - h, N_DEV)
            c_ccw = lax.rem(my + 2 + h, N_DEV)
            p0 = pltpu.make_async_copy(chunk_lo(c_cw), tmp_cw, dsems.at[0])
            p1 = pltpu.make_async_copy(chunk_hi(c_ccw), tmp_ccw, dsems.at[1])
            p0.start()
            p1.start()

            rdma_cw.wait_send()
            rdma_ccw.wait_send()
            if h < N_DEV - 2:
                pl.semaphore_signal(
                    credit_cw, inc=1,
                    device_id=(left,), device_id_type=pl.DeviceIdType.MESH,
                )
                pl.semaphore_signal(
                    credit_ccw, inc=1,
                    device_id=(right,), device_id_type=pl.DeviceIdType.MESH,
                )

            rdma_cw.wait_recv()
            rdma_ccw.wait_recv()
            p0.wait()
            p1.wait()

            cw_slots[r] = cw_slots[r] + tmp_cw[...]
            ccw_slots[r] = ccw_slots[r] + tmp_ccw[...]

        ssq = (jnp.sum(cw_slots[1] * cw_slots[1], axis=-1, keepdims=True)
               + jnp.sum(ccw_slots[1] * ccw_slots[1], axis=-1, keepdims=True))
        inv = lax.rsqrt(ssq / D + EPS)
        cw_slots[0] = cw_slots[1] * inv * g_ref[:, 0:DH]
        ccw_slots[0] = ccw_slots[1] * inv * g_ref[:, DH:D]

        o0 = pltpu.make_async_copy(cw_slots.at[0], o_ref.at[:, pl.ds(0, DH)],
                                   dsems.at[0])
        o1 = pltpu.make_async_copy(ccw_slots.at[0], o_ref.at[:, pl.ds(DH, DH)],
                                   dsems.at[1])
        o0.start()
        o1.start()
        o0.wait()
        o1.wait()

    return pl.pallas_call(
        body,
        out_shape=jax.ShapeDtypeStruct((M_PER, D), jnp.float32),
        in_specs=[
            pl.BlockSpec(memory_space=pl.ANY),
            pl.BlockSpec(memory_space=pltpu.VMEM),
        ],
        out_specs=pl.BlockSpec(memory_space=pltpu.MemorySpace.HBM),
        scratch_shapes=[
            pltpu.VMEM((2, M_PER, DH), jnp.float32),
            pltpu.VMEM((2, M_PER, DH), jnp.float32),
            pltpu.VMEM((M_PER, DH), jnp.float32),
            pltpu.VMEM((M_PER, DH), jnp.float32),
            pltpu.SemaphoreType.DMA((2,)),
            pltpu.SemaphoreType.DMA((2, 2)),
            pltpu.SemaphoreType.DMA((2,)),
            pltpu.SemaphoreType.REGULAR,
            pltpu.SemaphoreType.REGULAR,
        ],
        compiler_params=pltpu.CompilerParams(
            collective_id=0, vmem_limit_bytes=63 * 1024 * 1024),
    )(partial, gamma2d)


# device time: 297726 ns/iter; 2.0542x vs baseline; 1.0402x over previous
import jax
import jax.numpy as jnp
from jax import lax
from jax.experimental import pallas as pl
from jax.experimental.pallas import tpu as pltpu

N_DEV = 4
M_PER = 2048
D = 2048
DH = D // 2
RH = M_PER // 2
EPS = 1e-6


def kernel(partial, gamma):
    gamma2d = gamma.reshape(1, D)

    def body(p_ref, g_ref, o_ref,
             cw_slots, ccw_slots, tmp_cw, tmp_ccw,
             send_sems, recv_sems, dsems, credit_cw, credit_ccw):
        my = lax.axis_index("i")
        left = lax.rem(my + N_DEV - 1, N_DEV)
        right = lax.rem(my + 1, N_DEV)

        def chunk_lo(c):
            return p_ref.at[0, pl.ds(c * M_PER, M_PER), pl.ds(0, DH)]

        def chunk_hi(c):
            return p_ref.at[0, pl.ds(c * M_PER, M_PER), pl.ds(DH, DH)]

        def rdma(dir_idx, h, rh):
            slots = cw_slots if dir_idx == 0 else ccw_slots
            return pltpu.make_async_remote_copy(
                src_ref=slots.at[h % 2, pl.ds(rh * RH, RH), :],
                dst_ref=slots.at[(h + 1) % 2, pl.ds(rh * RH, RH), :],
                send_sem=send_sems.at[dir_idx, rh],
                recv_sem=recv_sems.at[dir_idx, (h + 1) % 2, rh],
                device_id=(right if dir_idx == 0 else left,),
                device_id_type=pl.DeviceIdType.MESH,
            )

        s0 = pltpu.make_async_copy(chunk_lo(left), cw_slots.at[0], dsems.at[0])
        s1 = pltpu.make_async_copy(chunk_hi(right), ccw_slots.at[0],
                                   dsems.at[1])
        s0.start()
        s1.start()

        barrier = pltpu.get_barrier_semaphore()
        for nbr in (left, right):
            pl.semaphore_signal(
                barrier, inc=1,
                device_id=(nbr,), device_id_type=pl.DeviceIdType.MESH,
            )
        pl.semaphore_wait(barrier, 2)
        s0.wait()
        s1.wait()

        for rh in (0, 1):
            rdma(0, 0, rh).start()
            rdma(1, 0, rh).start()
        c_cw = lax.rem(my + 2 * N_DEV - 2, N_DEV)
        c_ccw = lax.rem(my + 2, N_DEV)
        p0 = pltpu.make_async_copy(chunk_lo(c_cw), tmp_cw, dsems.at[0])
        p1 = pltpu.make_async_copy(chunk_hi(c_ccw), tmp_ccw, dsems.at[1])
        p0.start()
        p1.start()

        out_copies = []
        for h in range(N_DEV - 1):
            r = (h + 1) % 2
            rows = lambda rh: pl.ds(rh * RH, RH)
            for rh in (0, 1):
                d_cw = rdma(0, h, rh)
                d_ccw = rdma(1, h, rh)
                d_cw.wait_send()
                d_ccw.wait_send()
                if h < N_DEV - 2:
                    pl.semaphore_signal(
                        credit_cw.at[rh], inc=1,
                        device_id=(left,),
                        device_id_type=pl.DeviceIdType.MESH,
                    )
                    pl.semaphore_signal(
                        credit_ccw.at[rh], inc=1,
                        device_id=(right,),
                        device_id_type=pl.DeviceIdType.MESH,
                    )
                d_cw.wait_recv()
                d_ccw.wait_recv()
                if rh == 0:
                    p0.wait()
                    p1.wait()
                cw_slots[r, rows(rh), :] = (cw_slots[r, rows(rh), :]
                                            + tmp_cw[rows(rh), :])
                ccw_slots[r, rows(rh), :] = (ccw_slots[r, rows(rh), :]
                                             + tmp_ccw[rows(rh), :])
                if h < N_DEV - 2:
                    pl.semaphore_wait(credit_cw.at[rh], 1)
                    pl.semaphore_wait(credit_ccw.at[rh], 1)
                    rdma(0, h + 1, rh).start()
                    rdma(1, h + 1, rh).start()
                else:
                    a = cw_slots[1, rows(rh), :]
                    b = ccw_slots[1, rows(rh), :]
                    ssq = (jnp.sum(a * a, axis=-1, keepdims=True)
                           + jnp.sum(b * b, axis=-1, keepdims=True))
                    inv = lax.rsqrt(ssq / D + EPS)
                    cw_slots[0, rows(rh), :] = a * inv * g_ref[:, 0:DH]
                    ccw_slots[0, rows(rh), :] = b * inv * g_ref[:, DH:D]
                    oc = pltpu.make_async_copy(
                        cw_slots.at[0, rows(rh), :],
                        o_ref.at[rows(rh), pl.ds(0, DH)],
                        dsems.at[2 * rh])
                    od = pltpu.make_async_copy(
                        ccw_slots.at[0, rows(rh), :],
                        o_ref.at[rows(rh), pl.ds(DH, DH)],
                        dsems.at[2 * rh + 1])
                    oc.start()
                    od.start()
                    out_copies += [oc, od]
                    if rh == 1:
                        for c in out_copies:
                            c.wait()
            if h < N_DEV - 2:
                c_cw = lax.rem(my + 2 * N_DEV - 3 - h, N_DEV)
                c_ccw = lax.rem(my + 3 + h, N_DEV)
                p0 = pltpu.make_async_copy(chunk_lo(c_cw), tmp_cw,
                                           dsems.at[0])
                p1 = pltpu.make_async_copy(chunk_hi(c_ccw), tmp_ccw,
                                           dsems.at[1])
                p0.start()
                p1.start()

    return pl.pallas_call(
        body,
        out_shape=jax.ShapeDtypeStruct((M_PER, D), jnp.float32),
        in_specs=[
            pl.BlockSpec(memory_space=pl.ANY),
            pl.BlockSpec(memory_space=pltpu.VMEM),
        ],
        out_specs=pl.BlockSpec(memory_space=pltpu.MemorySpace.HBM),
        scratch_shapes=[
            pltpu.VMEM((2, M_PER, DH), jnp.float32),
            pltpu.VMEM((2, M_PER, DH), jnp.float32),
            pltpu.VMEM((M_PER, DH), jnp.float32),
            pltpu.VMEM((M_PER, DH), jnp.float32),
            pltpu.SemaphoreType.DMA((2, 2)),
            pltpu.SemaphoreType.DMA((2, 2, 2)),
            pltpu.SemaphoreType.DMA((4,)),
            pltpu.SemaphoreType.REGULAR((2,)),
            pltpu.SemaphoreType.REGULAR((2,)),
        ],
        compiler_params=pltpu.CompilerParams(
            collective_id=0, vmem_limit_bytes=63 * 1024 * 1024),
    )(partial, gamma2d)


# device time: 292366 ns/iter; 2.0919x vs baseline; 1.0183x over previous
import jax
import jax.numpy as jnp
from jax import lax
from jax.experimental import pallas as pl
from jax.experimental.pallas import tpu as pltpu

N_DEV = 4
M_PER = 2048
D = 2048
DH = D // 2
NSUB = 4
RS = M_PER // NSUB
EPS = 1e-6


def kernel(partial, gamma):
    gamma2d = gamma.reshape(1, D)

    def body(p_ref, g_ref, o_ref,
             cw_slots, ccw_slots, tmp_cw, tmp_ccw,
             send_sems, recv_sems, dsems, seed_sems,
             credit_cw, credit_ccw):
        my = lax.axis_index("i")
        left = lax.rem(my + N_DEV - 1, N_DEV)
        right = lax.rem(my + 1, N_DEV)

        def rows(q):
            return pl.ds(q * RS, RS)

        def chunk_lo(c, q):
            return p_ref.at[0, pl.ds(c * M_PER + q * RS, RS), pl.ds(0, DH)]

        def chunk_hi(c, q):
            return p_ref.at[0, pl.ds(c * M_PER + q * RS, RS), pl.ds(DH, DH)]

        def rdma(dir_idx, h, q):
            slots = cw_slots if dir_idx == 0 else ccw_slots
            return pltpu.make_async_remote_copy(
                src_ref=slots.at[h % 2, rows(q), :],
                dst_ref=slots.at[(h + 1) % 2, rows(q), :],
                send_sem=send_sems.at[dir_idx, q],
                recv_sem=recv_sems.at[dir_idx, (h + 1) % 2, q],
                device_id=(right if dir_idx == 0 else left,),
                device_id_type=pl.DeviceIdType.MESH,
            )

        seeds = []
        for q in range(NSUB):
            sc = pltpu.make_async_copy(chunk_lo(left, q),
                                       cw_slots.at[0, rows(q), :],
                                       seed_sems.at[0, q])
            sd = pltpu.make_async_copy(chunk_hi(right, q),
                                       ccw_slots.at[0, rows(q), :],
                                       seed_sems.at[1, q])
            sc.start()
            sd.start()
            seeds.append((sc, sd))

        barrier = pltpu.get_barrier_semaphore()
        for nbr in (left, right):
            pl.semaphore_signal(
                barrier, inc=1,
                device_id=(nbr,), device_id_type=pl.DeviceIdType.MESH,
            )
        pl.semaphore_wait(barrier, 2)

        for q in range(NSUB):
            seeds[q][0].wait()
            seeds[q][1].wait()
            rdma(0, 0, q).start()
            rdma(1, 0, q).start()

        c_cw = lax.rem(my + 2 * N_DEV - 2, N_DEV)
        c_ccw = lax.rem(my + 2, N_DEV)
        p0 = pltpu.make_async_copy(
            p_ref.at[0, pl.ds(c_cw * M_PER, M_PER), pl.ds(0, DH)],
            tmp_cw, dsems.at[0])
        p1 = pltpu.make_async_copy(
            p_ref.at[0, pl.ds(c_ccw * M_PER, M_PER), pl.ds(DH, DH)],
            tmp_ccw, dsems.at[1])
        p0.start()
        p1.start()

        out_copies = []
        for h in range(N_DEV - 1):
            r = (h + 1) % 2
            for q in range(NSUB):
                d_cw = rdma(0, h, q)
                d_ccw = rdma(1, h, q)
                d_cw.wait_send()
                d_ccw.wait_send()
                if h < N_DEV - 2:
                    pl.semaphore_signal(
                        credit_cw.at[q], inc=1,
                        device_id=(left,),
                        device_id_type=pl.DeviceIdType.MESH,
                    )
                    pl.semaphore_signal(
                        credit_ccw.at[q], inc=1,
                        device_id=(right,),
                        device_id_type=pl.DeviceIdType.MESH,
                    )
                d_cw.wait_recv()
                d_ccw.wait_recv()
                if q == 0:
                    p0.wait()
                    p1.wait()
                cw_slots[r, rows(q), :] = (cw_slots[r, rows(q), :]
                                           + tmp_cw[rows(q), :])
                ccw_slots[r, rows(q), :] = (ccw_slots[r, rows(q), :]
                                            + tmp_ccw[rows(q), :])
                if h < N_DEV - 2:
                    pl.semaphore_wait(credit_cw.at[q], 1)
                    pl.semaphore_wait(credit_ccw.at[q], 1)
                    rdma(0, h + 1, q).start()
                    rdma(1, h + 1, q).start()
                else:
                    a = cw_slots[1, rows(q), :]
                    b = ccw_slots[1, rows(q), :]
                    ssq = (jnp.sum(a * a, axis=-1, keepdims=True)
                           + jnp.sum(b * b, axis=-1, keepdims=True))
                    inv = lax.rsqrt(ssq / D + EPS)
                    cw_slots[0, rows(q), :] = a * inv * g_ref[:, 0:DH]
                    ccw_slots[0, rows(q), :] = b * inv * g_ref[:, DH:D]
                    oc = pltpu.make_async_copy(
                        cw_slots.at[0, rows(q), :],
                        o_ref.at[rows(q), pl.ds(0, DH)],
                        seed_sems.at[0, q])
                    od = pltpu.make_async_copy(
                        ccw_slots.at[0, rows(q), :],
                        o_ref.at[rows(q), pl.ds(DH, DH)],
                        seed_sems.at[1, q])
                    oc.start()
                    od.start()
                    out_copies += [oc, od]
            if h < N_DEV - 2:
                c_cw = lax.rem(my + 2 * N_DEV - 3 - h, N_DEV)
                c_ccw = lax.rem(my + 3 + h, N_DEV)
                p0 = pltpu.make_async_copy(
                    p_ref.at[0, pl.ds(c_cw * M_PER, M_PER), pl.ds(0, DH)],
                    tmp_cw, dsems.at[0])
                p1 = pltpu.make_async_copy(
                    p_ref.at[0, pl.ds(c_ccw * M_PER, M_PER), pl.ds(DH, DH)],
                    tmp_ccw, dsems.at[1])
                p0.start()
                p1.start()

        for c in out_copies:
            c.wait()

    return pl.pallas_call(
        body,
        out_shape=jax.ShapeDtypeStruct((M_PER, D), jnp.float32),
        in_specs=[
            pl.BlockSpec(memory_space=pl.ANY),
            pl.BlockSpec(memory_space=pltpu.VMEM),
        ],
        out_specs=pl.BlockSpec(memory_space=pltpu.MemorySpace.HBM),
        scratch_shapes=[
            pltpu.VMEM((2, M_PER, DH), jnp.float32),
            pltpu.VMEM((2, M_PER, DH), jnp.float32),
            pltpu.VMEM((M_PER, DH), jnp.float32),
            pltpu.VMEM((M_PER, DH), jnp.float32),
            pltpu.SemaphoreType.DMA((2, NSUB)),
            pltpu.SemaphoreType.DMA((2, 2, NSUB)),
            pltpu.SemaphoreType.DMA((2,)),
            pltpu.SemaphoreType.DMA((2, NSUB)),
            pltpu.SemaphoreType.REGULAR((NSUB,)),
            pltpu.SemaphoreType.REGULAR((NSUB,)),
        ],
        compiler_params=pltpu.CompilerParams(
            collective_id=0, vmem_limit_bytes=63 * 1024 * 1024),
    )(partial, gamma2d)
